# baseline (speedup 1.0000x reference)
import jax
import jax.numpy as jnp
from jax.experimental import pallas as pl
from jax.experimental.pallas import tpu as pltpu

_BS = 512


def _body(x_ref, emb_ref, out_ref, carry_ref):
    j = pl.program_id(1)
    T = emb_ref.shape[0]
    top = jnp.where(j == 0, emb_ref[...], carry_ref[0:T])
    new_carry = x_ref[0, _BS - T:_BS, :]
    out_ref[0, 0:T, :] = top
    out_ref[0, T:_BS, :] = x_ref[0, 0:_BS - T, :]
    carry_ref[0:T] = new_carry


def kernel(x, embed_table):
    B, S, D = x.shape
    T = embed_table.shape[0]
    S_out = S + T
    nj = pl.cdiv(S_out, _BS)
    nx = pl.cdiv(S, _BS)
    return pl.pallas_call(
        _body,
        grid=(B, nj),
        out_shape=jax.ShapeDtypeStruct((B, S_out, D), x.dtype),
        in_specs=[
            pl.BlockSpec((1, _BS, D), lambda b, j: (b, jnp.minimum(j, nx - 1), 0)),
            pl.BlockSpec((T, D), lambda b, j: (0, 0)),
        ],
        out_specs=pl.BlockSpec((1, _BS, D), lambda b, j: (b, j, 0)),
        scratch_shapes=[pltpu.VMEM((8, D), x.dtype)],
        compiler_params=pltpu.CompilerParams(
            dimension_semantics=("arbitrary", "arbitrary"),
        ),
    )(x, embed_table)
